# Initial kernel scaffold; baseline (speedup 1.0000x reference)
import jax
import jax.numpy as jnp
from jax.experimental import pallas as pl


def kernel(x_ind, x_org, x_ext, edge_index_ind_org, edge_index_org_ind, edge_index_ext_ind, edge_index_ext_org, W_ind, b_ind, W_org, b_org, W_ext, b_ext, att_src_io, att_dst_io, att_src_oi, att_dst_oi, att_src_ei, att_dst_ei, att_src_eo, att_dst_eo, k_W, k_b, q, lin_ind_W, lin_ind_b, lin_org_W, lin_org_b):
    raise NotImplementedError("write your pallas kernel here")


# speedup vs baseline: 38.8064x; 38.8064x over previous
import functools

import jax
import jax.numpy as jnp
from jax import lax
from jax.experimental import pallas as pl
from jax.experimental.pallas import tpu as pltpu
from jax.experimental.pallas import tpu_sc as plsc

_LANES = 16
_NSUB = 16
_NCORE = 2
_NW = _NSUB * _NCORE
_CHUNK = 1024
_G = _CHUNK // 128
_ZR = 256


def _round_up(x, m):
    return (x + m - 1) // m * m


def _block_rows(n):
    for b in (1024, 1000, 512, 500, 256, 200, 128, 100, 64, 40, 32, 16, 8):
        if n % b == 0:
            return b
    return 8


def _proj(x, w_aug, b_aug):
    n, d = x.shape
    bn = _block_rows(n)
    nblk = n // bn + 1

    def body(x_ref, w_ref, b_ref, o_ref):
        o_ref[...] = (
            jnp.dot(x_ref[...], w_ref[...], preferred_element_type=jnp.float32)
            + b_ref[...]
        )

    return pl.pallas_call(
        body,
        grid=(nblk,),
        in_specs=[
            pl.BlockSpec((bn, d), lambda i: (jnp.minimum(i, n // bn - 1), 0)),
            pl.BlockSpec((d, 16), lambda i: (0, 0)),
            pl.BlockSpec((1, 16), lambda i: (0, 0)),
        ],
        out_specs=pl.BlockSpec((bn, 16), lambda i: (i, 0)),
        out_shape=jax.ShapeDtypeStruct((nblk * bn, 16), jnp.float32),
    )(x, w_aug, b_aug)


def _aug_table(x, w, b, atts):
    d = w.shape[0]
    cols = [w, jnp.zeros((d, 1), jnp.float32)]
    bcols = [b, jnp.ones((1,), jnp.float32)]
    for a in atts:
        cols.append(w @ a[:, None])
        bcols.append((b @ a)[None])
    npad = 16 - 9 - len(atts)
    cols.append(jnp.zeros((d, npad), jnp.float32))
    bcols.append(jnp.zeros((npad,), jnp.float32))
    w_aug = jnp.concatenate(cols, axis=1)
    b_aug = jnp.concatenate(bcols)[None, :]
    return _proj(x, w_aug, b_aug)


def _edge_pass(h_src, h_dst, src_ids, dst_ids, n_dst_pad, col_s, col_d):
    e_pad = src_ids.shape[0]
    per_w = e_pad // _NW
    iters = per_w // _CHUNK
    stripe = n_dst_pad // _NSUB
    row_chunks = [(k * _CHUNK, _CHUNK) for k in range(stripe // _CHUNK)]
    if stripe % _CHUNK:
        row_chunks.append((stripe // _CHUNK * _CHUNK, stripe % _CHUNK))

    mesh = plsc.VectorSubcoreMesh(core_axis_name="c", subcore_axis_name="s")

    @functools.partial(
        pl.kernel,
        out_type=[
            jax.ShapeDtypeStruct((_NCORE, n_dst_pad, 8), jnp.float32),
            jax.ShapeDtypeStruct((_NCORE, n_dst_pad), jnp.float32),
        ],
        mesh=mesh,
        compiler_params=pltpu.CompilerParams(
            needs_layout_passes=False, use_tc_tiling_on_sc=False),
        scratch_types=[
            pltpu.VMEM((_G, 128), jnp.int32),
            pltpu.VMEM((_G, 128), jnp.int32),
            pltpu.VMEM((_CHUNK, 16), jnp.float32),
            pltpu.VMEM((_CHUNK, 16), jnp.float32),
            pltpu.VMEM((_CHUNK, 8), jnp.float32),
            pltpu.VMEM((_CHUNK,), jnp.float32),
            pltpu.VMEM_SHARED((n_dst_pad, 8), jnp.float32),
            pltpu.VMEM_SHARED((n_dst_pad,), jnp.float32),
            pltpu.SemaphoreType.DMA,
        ],
    )
    def body(hs, hd, si, di, onum, oden, src2, dst2, srows, drows,
             msg, ebuf, accn, accd, sem):
        c = lax.axis_index("c")
        s = lax.axis_index("s")
        wid = c * _NSUB + s

        def zmsg(i, _):
            rows = i * 2 + lax.iota(jnp.int32, _LANES) // 8
            cols = lax.iota(jnp.int32, _LANES) % 8
            plsc.store_scatter(msg, [rows, cols], jnp.zeros((_LANES,),
                                                            jnp.float32))
            return 0

        lax.fori_loop(0, _CHUNK // 2, zmsg, 0)

        def zebuf(i, _):
            ebuf[pl.ds(i * _LANES, _LANES)] = jnp.zeros((_LANES,), jnp.float32)
            return 0

        lax.fori_loop(0, _CHUNK // _LANES, zebuf, 0)
        for off, sz in row_chunks:
            pltpu.sync_copy(msg.at[pl.ds(0, sz)],
                            accn.at[pl.ds(s * stripe + off, sz)])
            pltpu.sync_copy(ebuf.at[pl.ds(0, sz)],
                            accd.at[pl.ds(s * stripe + off, sz)])
        plsc.subcore_barrier()

        base_w = wid * per_w

        def chunk_body(t, _):
            base = base_w + t * _CHUNK
            for g in range(_G):
                pltpu.sync_copy(si.at[pl.ds(base + g * 128, 128)], src2.at[g])
                pltpu.sync_copy(di.at[pl.ds(base + g * 128, 128)], dst2.at[g])
            cps = []
            for g in range(_G):
                cps.append(pltpu.async_copy(
                    hs.at[src2.at[g]], srows.at[pl.ds(g * 128, 128)], sem))
                cps.append(pltpu.async_copy(
                    hd.at[dst2.at[g]], drows.at[pl.ds(g * 128, 128)], sem))
            for cp in cps:
                cp.wait()

            def grp(i, _):
                rows = lax.iota(jnp.int32, _LANES) + i * _LANES
                a_s = plsc.load_gather(
                    srows, [rows, jnp.full((_LANES,), col_s, jnp.int32)])
                a_d = plsc.load_gather(
                    drows, [rows, jnp.full((_LANES,), col_d, jnp.int32)])
                al = a_s + a_d
                al = jnp.where(al > 0, al, 0.2 * al)
                e = jnp.exp(al)
                ebuf[pl.ds(i * _LANES, _LANES)] = e
                for j in range(8):
                    cj = jnp.full((_LANES,), j, jnp.int32)
                    v = plsc.load_gather(srows, [rows, cj]) * e
                    plsc.store_scatter(msg, [rows, cj], v)
                return 0

            lax.fori_loop(0, _CHUNK // _LANES, grp, 0)

            for g in range(_G):
                pltpu.sync_copy(msg.at[pl.ds(g * 128, 128)],
                                accn.at[dst2.at[g]], add=True)
                pltpu.sync_copy(ebuf.at[pl.ds(g * 128, 128)],
                                accd.at[dst2.at[g]], add=True)
            return 0

        lax.fori_loop(0, iters, chunk_body, 0)
        plsc.subcore_barrier()

        for off, sz in row_chunks:
            o = s * stripe + off
            pltpu.sync_copy(accn.at[pl.ds(o, sz)], onum.at[c, pl.ds(o, sz)])
            pltpu.sync_copy(accd.at[pl.ds(o, sz)], oden.at[c, pl.ds(o, sz)])

    return body(h_src, h_dst, src_ids, dst_ids)


def _pad_edges(eidx, n_dst):
    e = eidx.shape[1]
    e_pad = _round_up(e, _NW * _CHUNK)
    pad = e_pad - e
    src = jnp.concatenate([eidx[0], jnp.zeros((pad,), jnp.int32)])
    dst = jnp.concatenate([eidx[1], jnp.full((pad,), n_dst, jnp.int32)])
    return src, dst


def _combine(pa_num, pa_den, pb_num, pb_den, k_w, k_b, n_dst):
    n_pad = pa_num.shape[1]
    bn = 1024

    def body(pan_ref, pad_ref, pbn_ref, pbd_ref, kw_ref, kb_ref,
             oa_ref, ob_ref, sums_ref):
        i = pl.program_id(0)

        @pl.when(i == 0)
        def _():
            sums_ref[...] = jnp.zeros_like(sums_ref)

        na = pan_ref[0] + pan_ref[1]
        nb = pbn_ref[0] + pbn_ref[1]
        da = (pad_ref[0] + pad_ref[1])[:, None]
        db = (pbd_ref[0] + pbd_ref[1])[:, None]
        oa = jnp.maximum(na / (da + 1e-16), 0.0)
        ob = jnp.maximum(nb / (db + 1e-16), 0.0)
        oa_ref[...] = oa
        ob_ref[...] = ob
        rid = jax.lax.broadcasted_iota(jnp.int32, (bn, 8), 0) + i * bn
        valid = rid < n_dst
        ta = jnp.tanh(
            jnp.dot(oa, kw_ref[...], preferred_element_type=jnp.float32)
            + kb_ref[...])
        tb = jnp.tanh(
            jnp.dot(ob, kw_ref[...], preferred_element_type=jnp.float32)
            + kb_ref[...])
        ta = jnp.where(valid, ta, 0.0)
        tb = jnp.where(valid, tb, 0.0)
        part = jnp.stack([jnp.sum(ta, axis=0), jnp.sum(tb, axis=0)], axis=0)
        sums_ref[...] += part

    return pl.pallas_call(
        body,
        grid=(n_pad // bn,),
        in_specs=[
            pl.BlockSpec((2, bn, 8), lambda i: (0, i, 0)),
            pl.BlockSpec((2, bn), lambda i: (0, i)),
            pl.BlockSpec((2, bn, 8), lambda i: (0, i, 0)),
            pl.BlockSpec((2, bn), lambda i: (0, i)),
            pl.BlockSpec((8, 8), lambda i: (0, 0)),
            pl.BlockSpec((1, 8), lambda i: (0, 0)),
        ],
        out_specs=[
            pl.BlockSpec((bn, 8), lambda i: (i, 0)),
            pl.BlockSpec((bn, 8), lambda i: (i, 0)),
            pl.BlockSpec((2, 8), lambda i: (0, 0)),
        ],
        out_shape=[
            jax.ShapeDtypeStruct((n_pad, 8), jnp.float32),
            jax.ShapeDtypeStruct((n_pad, 8), jnp.float32),
            jax.ShapeDtypeStruct((2, 8), jnp.float32),
        ],
    )(pa_num, pa_den, pb_num, pb_den, k_w, k_b)


def _head(oa, ob, sums, q2, lw, lb, n_dst):
    n_pad = oa.shape[0]
    bn = 1024

    def body(oa_ref, ob_ref, sums_ref, q_ref, lw_ref, lb_ref, o_ref):
        inv_n = 1.0 / n_dst
        s_a = jnp.sum(q_ref[0, :] * sums_ref[0, :]) * inv_n
        s_b = jnp.sum(q_ref[0, :] * sums_ref[1, :]) * inv_n
        m = jnp.maximum(s_a, s_b)
        ea = jnp.exp(s_a - m)
        eb = jnp.exp(s_b - m)
        w_a = ea / (ea + eb)
        w_b = eb / (ea + eb)
        z = w_a * oa_ref[...] + w_b * ob_ref[...]
        logit = jnp.sum(z * lw_ref[...], axis=1, keepdims=True) + lb_ref[0, 0]
        o_ref[...] = jax.nn.sigmoid(logit)

    return pl.pallas_call(
        body,
        grid=(n_pad // bn,),
        in_specs=[
            pl.BlockSpec((bn, 8), lambda i: (i, 0)),
            pl.BlockSpec((bn, 8), lambda i: (i, 0)),
            pl.BlockSpec((2, 8), lambda i: (0, 0)),
            pl.BlockSpec((1, 8), lambda i: (0, 0)),
            pl.BlockSpec((1, 8), lambda i: (0, 0)),
            pl.BlockSpec((1, 1), lambda i: (0, 0)),
        ],
        out_specs=pl.BlockSpec((bn, 1), lambda i: (i, 0)),
        out_shape=jax.ShapeDtypeStruct((n_pad, 1), jnp.float32),
    )(oa, ob, sums, q2, lw, lb)


def kernel(x_ind, x_org, x_ext, edge_index_ind_org, edge_index_org_ind,
           edge_index_ext_ind, edge_index_ext_org, W_ind, b_ind, W_org, b_org,
           W_ext, b_ext, att_src_io, att_dst_io, att_src_oi, att_dst_oi,
           att_src_ei, att_dst_ei, att_src_eo, att_dst_eo, k_W, k_b, q,
           lin_ind_W, lin_ind_b, lin_org_W, lin_org_b):
    n_ind = x_ind.shape[0]
    n_org = x_org.shape[0]

    t_ind = _aug_table(x_ind, W_ind, b_ind,
                       [att_src_io.reshape(-1), att_dst_oi.reshape(-1),
                        att_dst_ei.reshape(-1)])
    t_org = _aug_table(x_org, W_org, b_org,
                       [att_src_oi.reshape(-1), att_dst_io.reshape(-1),
                        att_dst_eo.reshape(-1)])
    t_ext = _aug_table(x_ext, W_ext, b_ext,
                       [att_src_ei.reshape(-1), att_src_eo.reshape(-1)])

    n_ind_pad = _round_up(n_ind + 1, _NSUB * _ZR)
    n_org_pad = _round_up(n_org + 1, _NSUB * _ZR)

    src_io, dst_io = _pad_edges(edge_index_ind_org, n_org)
    src_oi, dst_oi = _pad_edges(edge_index_org_ind, n_ind)
    src_ei, dst_ei = _pad_edges(edge_index_ext_ind, n_ind)
    src_eo, dst_eo = _pad_edges(edge_index_ext_org, n_org)

    p_io = _edge_pass(t_ind, t_org, src_io, dst_io, n_org_pad, 9, 10)
    p_oi = _edge_pass(t_org, t_ind, src_oi, dst_oi, n_ind_pad, 9, 10)
    p_ei = _edge_pass(t_ext, t_ind, src_ei, dst_ei, n_ind_pad, 9, 11)
    p_eo = _edge_pass(t_ext, t_org, src_eo, dst_eo, n_org_pad, 10, 11)

    kb2 = k_b[None, :]
    q2 = q[None, :]

    out_oi, out_ei, sums_ind = _combine(p_oi[0], p_oi[1], p_ei[0], p_ei[1],
                                        k_W, kb2, n_ind)
    out_io, out_eo, sums_org = _combine(p_io[0], p_io[1], p_eo[0], p_eo[1],
                                        k_W, kb2, n_org)

    pred_ind = _head(out_oi, out_ei, sums_ind, q2,
                     lin_ind_W.reshape(1, -1), lin_ind_b.reshape(1, 1), n_ind)
    pred_org = _head(out_io, out_eo, sums_org, q2,
                     lin_org_W.reshape(1, -1), lin_org_b.reshape(1, 1), n_org)

    return (pred_ind[:n_ind, 0], pred_org[:n_org, 0])


# speedup vs baseline: 97.3045x; 2.5074x over previous
import functools

import jax
import jax.numpy as jnp
from jax import lax
from jax.experimental import pallas as pl
from jax.experimental.pallas import tpu as pltpu
from jax.experimental.pallas import tpu_sc as plsc

_LANES = 16
_NSUB = 16
_NCORE = 2
_NW = _NSUB * _NCORE
_CHUNK = 1024
_G = _CHUNK // 128
_ZR = 256


def _round_up(x, m):
    return (x + m - 1) // m * m


def _block_rows(n):
    for b in (1024, 1000, 512, 500, 256, 200, 128, 100, 64, 40, 32, 16, 8):
        if n % b == 0:
            return b
    return 8


def _proj(x, w_aug, b_aug):
    n, d = x.shape
    bn = _block_rows(n)
    nblk = n // bn + 1

    def body(x_ref, w_ref, b_ref, o_ref):
        o_ref[...] = (
            jnp.dot(x_ref[...], w_ref[...], preferred_element_type=jnp.float32)
            + b_ref[...]
        )

    return pl.pallas_call(
        body,
        grid=(nblk,),
        in_specs=[
            pl.BlockSpec((bn, d), lambda i: (jnp.minimum(i, n // bn - 1), 0)),
            pl.BlockSpec((d, 16), lambda i: (0, 0)),
            pl.BlockSpec((1, 16), lambda i: (0, 0)),
        ],
        out_specs=pl.BlockSpec((bn, 16), lambda i: (i, 0)),
        out_shape=jax.ShapeDtypeStruct((nblk * bn, 16), jnp.float32),
    )(x, w_aug, b_aug)


def _aug_table(x, w, b, atts):
    d = w.shape[0]
    cols = [w, jnp.zeros((d, 1), jnp.float32)]
    bcols = [b, jnp.ones((1,), jnp.float32)]
    for a in atts:
        cols.append(w @ a[:, None])
        bcols.append((b @ a)[None])
    npad = 16 - 9 - len(atts)
    cols.append(jnp.zeros((d, npad), jnp.float32))
    bcols.append(jnp.zeros((npad,), jnp.float32))
    w_aug = jnp.concatenate(cols, axis=1)
    b_aug = jnp.concatenate(bcols)[None, :]
    return _proj(x, w_aug, b_aug)


def _edge_pass(h_src, h_dst, src_ids, dst_ids, n_dst_pad, col_s, col_d):
    e_pad = src_ids.shape[0]
    per_w = e_pad // _NW
    iters = per_w // _CHUNK
    stripe = n_dst_pad // _NSUB
    row_chunks = [(k * _CHUNK, _CHUNK) for k in range(stripe // _CHUNK)]
    if stripe % _CHUNK:
        row_chunks.append((stripe // _CHUNK * _CHUNK, stripe % _CHUNK))

    mesh = plsc.VectorSubcoreMesh(core_axis_name="c", subcore_axis_name="s")

    @functools.partial(
        pl.kernel,
        out_type=[
            jax.ShapeDtypeStruct((_NCORE, n_dst_pad, 8), jnp.float32),
            jax.ShapeDtypeStruct((_NCORE, n_dst_pad), jnp.float32),
        ],
        mesh=mesh,
        compiler_params=pltpu.CompilerParams(
            needs_layout_passes=False, use_tc_tiling_on_sc=False),
        scratch_types=[
            pltpu.VMEM((_G, 128), jnp.int32),
            pltpu.VMEM((_G, 128), jnp.int32),
            pltpu.VMEM((_CHUNK, 16), jnp.float32),
            pltpu.VMEM((_CHUNK, 16), jnp.float32),
            pltpu.VMEM((_CHUNK, 8), jnp.float32),
            pltpu.VMEM((_CHUNK,), jnp.float32),
            pltpu.VMEM_SHARED((n_dst_pad, 8), jnp.float32),
            pltpu.VMEM_SHARED((n_dst_pad,), jnp.float32),
            pltpu.SemaphoreType.DMA,
        ],
    )
    def body(hs, hd, si, di, onum, oden, src2, dst2, srows, drows,
             msg, ebuf, accn, accd, sem):
        c = lax.axis_index("c")
        s = lax.axis_index("s")
        wid = c * _NSUB + s

        def zmsg(i, _):
            rows = i * 2 + lax.iota(jnp.int32, _LANES) // 8
            cols = lax.iota(jnp.int32, _LANES) % 8
            plsc.store_scatter(msg, [rows, cols], jnp.zeros((_LANES,),
                                                            jnp.float32))
            return 0

        lax.fori_loop(0, _CHUNK // 2, zmsg, 0)

        def zebuf(i, _):
            ebuf[pl.ds(i * _LANES, _LANES)] = jnp.zeros((_LANES,), jnp.float32)
            return 0

        lax.fori_loop(0, _CHUNK // _LANES, zebuf, 0)
        for off, sz in row_chunks:
            pltpu.sync_copy(msg.at[pl.ds(0, sz)],
                            accn.at[pl.ds(s * stripe + off, sz)])
            pltpu.sync_copy(ebuf.at[pl.ds(0, sz)],
                            accd.at[pl.ds(s * stripe + off, sz)])
        plsc.subcore_barrier()

        base_w = wid * per_w

        def chunk_body(t, _):
            rbase = base_w // 128 + t * _G
            pltpu.sync_copy(si.at[pl.ds(rbase, _G)], src2)
            pltpu.sync_copy(di.at[pl.ds(rbase, _G)], dst2)
            cps = []
            for g in range(_G):
                cps.append(pltpu.async_copy(
                    hs.at[src2.at[g]], srows.at[pl.ds(g * 128, 128)], sem))
                cps.append(pltpu.async_copy(
                    hd.at[dst2.at[g]], drows.at[pl.ds(g * 128, 128)], sem))
            for cp in cps:
                cp.wait()

            def grp(i, _):
                rows = lax.iota(jnp.int32, _LANES) + i * _LANES
                a_s = plsc.load_gather(
                    srows, [rows, jnp.full((_LANES,), col_s, jnp.int32)])
                a_d = plsc.load_gather(
                    drows, [rows, jnp.full((_LANES,), col_d, jnp.int32)])
                al = a_s + a_d
                al = jnp.where(al > 0, al, 0.2 * al)
                e = jnp.exp(al)
                ebuf[pl.ds(i * _LANES, _LANES)] = e
                for j in range(8):
                    cj = jnp.full((_LANES,), j, jnp.int32)
                    v = plsc.load_gather(srows, [rows, cj]) * e
                    plsc.store_scatter(msg, [rows, cj], v)
                return 0

            lax.fori_loop(0, _CHUNK // _LANES, grp, 0)

            scs = []
            for g in range(_G):
                scs.append(pltpu.async_copy(
                    msg.at[pl.ds(g * 128, 128)], accn.at[dst2.at[g]], sem,
                    add=True))
                scs.append(pltpu.async_copy(
                    ebuf.at[pl.ds(g * 128, 128)], accd.at[dst2.at[g]], sem,
                    add=True))
            for cp in scs:
                cp.wait()
            return 0

        lax.fori_loop(0, iters, chunk_body, 0)
        plsc.subcore_barrier()

        for off, sz in row_chunks:
            o = s * stripe + off
            pltpu.sync_copy(accn.at[pl.ds(o, sz)], onum.at[c, pl.ds(o, sz)])
            pltpu.sync_copy(accd.at[pl.ds(o, sz)], oden.at[c, pl.ds(o, sz)])

    return body(h_src, h_dst, src_ids, dst_ids)


def _pad_edges(eidx, n_dst):
    e = eidx.shape[1]
    e_pad = _round_up(e, _NW * _CHUNK)
    pad = e_pad - e
    src = jnp.concatenate([eidx[0], jnp.zeros((pad,), jnp.int32)])
    dst = jnp.concatenate([eidx[1], jnp.full((pad,), n_dst, jnp.int32)])
    return src.reshape(e_pad // 128, 128), dst.reshape(e_pad // 128, 128)


def _combine(pa_num, pa_den, pb_num, pb_den, k_w, k_b, n_dst):
    n_pad = pa_num.shape[1]
    bn = 1024

    def body(pan_ref, pad_ref, pbn_ref, pbd_ref, kw_ref, kb_ref,
             oa_ref, ob_ref, sums_ref):
        i = pl.program_id(0)

        @pl.when(i == 0)
        def _():
            sums_ref[...] = jnp.zeros_like(sums_ref)

        na = pan_ref[0] + pan_ref[1]
        nb = pbn_ref[0] + pbn_ref[1]
        da = (pad_ref[0] + pad_ref[1])[:, None]
        db = (pbd_ref[0] + pbd_ref[1])[:, None]
        oa = jnp.maximum(na / (da + 1e-16), 0.0)
        ob = jnp.maximum(nb / (db + 1e-16), 0.0)
        oa_ref[...] = oa
        ob_ref[...] = ob
        rid = jax.lax.broadcasted_iota(jnp.int32, (bn, 8), 0) + i * bn
        valid = rid < n_dst
        ta = jnp.tanh(
            jnp.dot(oa, kw_ref[...], preferred_element_type=jnp.float32)
            + kb_ref[...])
        tb = jnp.tanh(
            jnp.dot(ob, kw_ref[...], preferred_element_type=jnp.float32)
            + kb_ref[...])
        ta = jnp.where(valid, ta, 0.0)
        tb = jnp.where(valid, tb, 0.0)
        part = jnp.stack([jnp.sum(ta, axis=0), jnp.sum(tb, axis=0)], axis=0)
        sums_ref[...] += part

    return pl.pallas_call(
        body,
        grid=(n_pad // bn,),
        in_specs=[
            pl.BlockSpec((2, bn, 8), lambda i: (0, i, 0)),
            pl.BlockSpec((2, bn), lambda i: (0, i)),
            pl.BlockSpec((2, bn, 8), lambda i: (0, i, 0)),
            pl.BlockSpec((2, bn), lambda i: (0, i)),
            pl.BlockSpec((8, 8), lambda i: (0, 0)),
            pl.BlockSpec((1, 8), lambda i: (0, 0)),
        ],
        out_specs=[
            pl.BlockSpec((bn, 8), lambda i: (i, 0)),
            pl.BlockSpec((bn, 8), lambda i: (i, 0)),
            pl.BlockSpec((2, 8), lambda i: (0, 0)),
        ],
        out_shape=[
            jax.ShapeDtypeStruct((n_pad, 8), jnp.float32),
            jax.ShapeDtypeStruct((n_pad, 8), jnp.float32),
            jax.ShapeDtypeStruct((2, 8), jnp.float32),
        ],
    )(pa_num, pa_den, pb_num, pb_den, k_w, k_b)


def _head(oa, ob, sums, q2, lw, lb, n_dst):
    n_pad = oa.shape[0]
    bn = 1024

    def body(oa_ref, ob_ref, sums_ref, q_ref, lw_ref, lb_ref, o_ref):
        inv_n = 1.0 / n_dst
        s_a = jnp.sum(q_ref[0, :] * sums_ref[0, :]) * inv_n
        s_b = jnp.sum(q_ref[0, :] * sums_ref[1, :]) * inv_n
        m = jnp.maximum(s_a, s_b)
        ea = jnp.exp(s_a - m)
        eb = jnp.exp(s_b - m)
        w_a = ea / (ea + eb)
        w_b = eb / (ea + eb)
        z = w_a * oa_ref[...] + w_b * ob_ref[...]
        logit = jnp.sum(z * lw_ref[...], axis=1, keepdims=True) + lb_ref[0, 0]
        o_ref[...] = jax.nn.sigmoid(logit)

    return pl.pallas_call(
        body,
        grid=(n_pad // bn,),
        in_specs=[
            pl.BlockSpec((bn, 8), lambda i: (i, 0)),
            pl.BlockSpec((bn, 8), lambda i: (i, 0)),
            pl.BlockSpec((2, 8), lambda i: (0, 0)),
            pl.BlockSpec((1, 8), lambda i: (0, 0)),
            pl.BlockSpec((1, 8), lambda i: (0, 0)),
            pl.BlockSpec((1, 1), lambda i: (0, 0)),
        ],
        out_specs=pl.BlockSpec((bn, 1), lambda i: (i, 0)),
        out_shape=jax.ShapeDtypeStruct((n_pad, 1), jnp.float32),
    )(oa, ob, sums, q2, lw, lb)


def kernel(x_ind, x_org, x_ext, edge_index_ind_org, edge_index_org_ind,
           edge_index_ext_ind, edge_index_ext_org, W_ind, b_ind, W_org, b_org,
           W_ext, b_ext, att_src_io, att_dst_io, att_src_oi, att_dst_oi,
           att_src_ei, att_dst_ei, att_src_eo, att_dst_eo, k_W, k_b, q,
           lin_ind_W, lin_ind_b, lin_org_W, lin_org_b):
    n_ind = x_ind.shape[0]
    n_org = x_org.shape[0]

    t_ind = _aug_table(x_ind, W_ind, b_ind,
                       [att_src_io.reshape(-1), att_dst_oi.reshape(-1),
                        att_dst_ei.reshape(-1)])
    t_org = _aug_table(x_org, W_org, b_org,
                       [att_src_oi.reshape(-1), att_dst_io.reshape(-1),
                        att_dst_eo.reshape(-1)])
    t_ext = _aug_table(x_ext, W_ext, b_ext,
                       [att_src_ei.reshape(-1), att_src_eo.reshape(-1)])

    n_ind_pad = _round_up(n_ind + 1, _NSUB * _ZR)
    n_org_pad = _round_up(n_org + 1, _NSUB * _ZR)

    src_io, dst_io = _pad_edges(edge_index_ind_org, n_org)
    src_oi, dst_oi = _pad_edges(edge_index_org_ind, n_ind)
    src_ei, dst_ei = _pad_edges(edge_index_ext_ind, n_ind)
    src_eo, dst_eo = _pad_edges(edge_index_ext_org, n_org)

    p_io = _edge_pass(t_ind, t_org, src_io, dst_io, n_org_pad, 9, 10)
    p_oi = _edge_pass(t_org, t_ind, src_oi, dst_oi, n_ind_pad, 9, 10)
    p_ei = _edge_pass(t_ext, t_ind, src_ei, dst_ei, n_ind_pad, 9, 11)
    p_eo = _edge_pass(t_ext, t_org, src_eo, dst_eo, n_org_pad, 10, 11)

    kb2 = k_b[None, :]
    q2 = q[None, :]

    out_oi, out_ei, sums_ind = _combine(p_oi[0], p_oi[1], p_ei[0], p_ei[1],
                                        k_W, kb2, n_ind)
    out_io, out_eo, sums_org = _combine(p_io[0], p_io[1], p_eo[0], p_eo[1],
                                        k_W, kb2, n_org)

    pred_ind = _head(out_oi, out_ei, sums_ind, q2,
                     lin_ind_W.reshape(1, -1), lin_ind_b.reshape(1, 1), n_ind)
    pred_org = _head(out_io, out_eo, sums_org, q2,
                     lin_org_W.reshape(1, -1), lin_org_b.reshape(1, 1), n_org)

    return (pred_ind[:n_ind, 0], pred_org[:n_org, 0])
